# baseline (speedup 1.0000x reference)
import jax
import jax.numpy as jnp
from jax.experimental import pallas as pl


def _bcast_kernel(w_ref, o_ref):
    w = w_ref[...]
    o_ref[...] = jnp.broadcast_to(w, o_ref.shape)


def kernel(mask, embed_weight):
    B, L = mask.shape
    D = embed_weight.shape[-1]
    BB = 1024
    grid = (L, B // BB)
    w3 = embed_weight.reshape(L, 1, D)
    return pl.pallas_call(
        _bcast_kernel,
        grid=grid,
        in_specs=[pl.BlockSpec((1, 1, D), lambda l, b: (l, 0, 0))],
        out_specs=pl.BlockSpec((1, BB, D), lambda l, b: (l, b, 0)),
        out_shape=jax.ShapeDtypeStruct((L, B, D), embed_weight.dtype),
    )(w3)


# speedup vs baseline: 1.7756x; 1.7756x over previous
import jax
import jax.numpy as jnp
from jax.experimental import pallas as pl
from jax.experimental.pallas import tpu as pltpu

_BB = 1024


def _copies(tile_ref, o_ref, sem_ref, l, p, nc, bb):
    return [
        pltpu.make_async_copy(
            tile_ref.at[p],
            o_ref.at[l, pl.ds(i * bb, bb), :],
            sem_ref.at[p, i],
        )
        for i in range(nc)
    ]


def _fanout_kernel(w_ref, o_ref, tile_ref, sem_ref):
    nbuf, bb, d = tile_ref.shape
    nc = sem_ref.shape[1]
    n_l = pl.num_programs(0)
    l = pl.program_id(0)
    p = jax.lax.rem(l, 2)

    @pl.when(l >= 2)
    def _():
        for c in _copies(tile_ref, o_ref, sem_ref, l - 2, p, nc, bb):
            c.wait()

    tile_ref[p] = jnp.broadcast_to(w_ref[0], (bb, d))
    for c in _copies(tile_ref, o_ref, sem_ref, l, p, nc, bb):
        c.start()

    @pl.when(l == n_l - 1)
    def _():
        for c in _copies(tile_ref, o_ref, sem_ref, l - 1, 1 - p, nc, bb):
            c.wait()
        for c in _copies(tile_ref, o_ref, sem_ref, l, p, nc, bb):
            c.wait()


def kernel(mask, embed_weight):
    B, L = mask.shape
    D = embed_weight.shape[-1]
    nc = B // _BB
    w3 = embed_weight.reshape(L, 1, D)
    return pl.pallas_call(
        _fanout_kernel,
        grid=(L,),
        in_specs=[pl.BlockSpec((1, 1, D), lambda l: (l, 0, 0))],
        out_specs=pl.BlockSpec(memory_space=pltpu.MemorySpace.HBM),
        out_shape=jax.ShapeDtypeStruct((L, B, D), embed_weight.dtype),
        scratch_shapes=[
            pltpu.VMEM((2, _BB, D), embed_weight.dtype),
            pltpu.SemaphoreType.DMA((2, nc)),
        ],
    )(w3)
